# baseline (speedup 1.0000x reference)
import functools

import jax
import jax.numpy as jnp
import numpy as np
from jax import lax
from jax.experimental import pallas as pl
from jax.experimental.pallas import tpu as pltpu
from jax.experimental.pallas import tpu_sc as plsc

KEHALF = 7.199822675975274
SR_CUTOFF = 5.0
LR_CUTOFF = 10.0
LR_CUTOFF2 = LR_CUTOFF * LR_CUTOFF

N_NODES = 100000
N_EDGES = 6400000

LANES = 16
ROW = 128
NC = 2
NS = 16
NW = NC * NS
CHUNK_ROWS = 16
WIN = 10240

N_ROWS = N_EDGES // ROW
BASE_ROWS = (N_ROWS // NW) // 8 * 8
LEFTOVER = N_ROWS - NW * BASE_ROWS
N_EXTRA8 = LEFTOVER // 8


def _dist_energy(d):
    x = d * d + 1.0
    y = lax.rsqrt(x)
    c2 = 1.0 / LR_CUTOFF2
    c1 = 2.0 / LR_CUTOFF
    e_ord = (1.0 / d - c1) + d * c2
    e_shl = y * (1.0 + x * c2) - c1
    t = jnp.minimum(d * (2.0 / SR_CUTOFF), 1.0)
    t3 = (t * t) * t
    q = (6.0 * t - 15.0) * t + 10.0
    diff = e_shl - e_ord
    e = e_shl - (t3 * q) * diff
    return jnp.where(d <= LR_CUTOFF, e, 0.0)


def _pack_body(d_ref, ii_ref, lo_ref, o_ref):
    fd = _dist_energy(d_ref[...])
    hi = lax.bitcast_convert_type(fd.astype(jnp.bfloat16),
                                  jnp.uint16).astype(jnp.int32) << 16
    off = jnp.minimum(ii_ref[...] - lo_ref[...], 65535)
    o_ref[...] = hi | off


def _build_sc_kernel(n_nodes, n_rows):
    assert n_rows % 8 == 0
    base_rows = (n_rows // NW) // 8 * 8
    leftover = n_rows - NW * base_rows
    assert leftover % 8 == 0
    n_extra8 = leftover // 8
    assert n_extra8 <= NW
    n_full = base_rows // CHUNK_ROWS
    n_pipe = n_full - (n_full % 2)
    tail_rows = base_rows - n_pipe * CHUNK_ROWS
    assert tail_rows % 8 == 0
    n_tail8 = tail_rows // 8
    slc = ((n_nodes + NS - 1) // NS + 7) // 8 * 8
    last_slc = n_nodes - (NS - 1) * slc
    assert last_slc > 0

    mesh = plsc.VectorSubcoreMesh(core_axis_name="c", subcore_axis_name="s")

    def body(pk_hbm, qa_hbm, ii_hbm, jj_hbm, lo_hbm, out_hbm,
             qa_v, pk_v, jj_v, e_v, iis_v, lo_v, win_v, acc_sh, sem_in):
        c = lax.axis_index("c")
        s = lax.axis_index("s")
        wid = s * NC + c
        row_base = wid * base_rows
        iota16 = lax.broadcasted_iota(jnp.int32, (LANES,), 0)
        is15 = iota16 == (LANES - 1)
        shift1 = jnp.minimum(iota16 + 1, LANES - 1)
        zeros16 = jnp.zeros((LANES,), jnp.float32)

        def lane_shift(x):
            return lax.gather(
                x, shift1[:, None],
                lax.GatherDimensionNumbers(
                    offset_dims=(), collapsed_slice_dims=(0,),
                    start_index_map=(0,)),
                slice_sizes=(1,),
                mode=lax.GatherScatterMode.PROMISE_IN_BOUNDS)

        def stage(row0, b):
            pltpu.async_copy(pk_hbm.at[pl.ds(row0, CHUNK_ROWS)], pk_v.at[b], sem_in)
            pltpu.async_copy(jj_hbm.at[pl.ds(row0, CHUNK_ROWS)], jj_v.at[b], sem_in)

        def wait_stage(b):
            pltpu.make_async_copy(pk_hbm.at[pl.ds(0, CHUNK_ROWS)], pk_v.at[b], sem_in).wait()
            pltpu.make_async_copy(jj_hbm.at[pl.ds(0, CHUNK_ROWS)], jj_v.at[b], sem_in).wait()

        if n_pipe > 0:
            stage(row_base, 0)

        def zwin_body(i, carry):
            win_v[pl.ds(i * LANES, LANES)] = zeros16
            return carry
        lax.fori_loop(0, WIN // LANES, zwin_body, 0)

        def zero_body(i, carry):
            qa_v[pl.ds(i * LANES, LANES)] = zeros16
            return carry
        lax.fori_loop(0, slc // LANES + 1, zero_body, 0)

        @pl.when(s < NS - 1)
        def _():
            pltpu.sync_copy(qa_v.at[pl.ds(0, slc)],
                            acc_sh.at[pl.ds(s * slc, slc)])

        @pl.when(s == NS - 1)
        def _():
            pltpu.sync_copy(qa_v.at[pl.ds(0, last_slc)],
                            acc_sh.at[pl.ds((NS - 1) * slc, last_slc)])

        pltpu.sync_copy(qa_hbm, qa_v)
        pltpu.sync_copy(lo_hbm, lo_v)
        plsc.subcore_barrier()

        def compute_rows_fast(b, nr):
            @plsc.parallel_loop(0, nr, unroll=2)
            def _(r):
                for v in range(ROW // LANES):
                    sl = pl.ds(v * LANES, LANES)
                    pk = pk_v[b, r, sl]
                    off = pk & 0xFFFF
                    fd = plsc.bitcast(pk & jnp.int32(-65536), jnp.float32)
                    qj = plsc.load_gather(qa_v, [jj_v[b, r, sl]])
                    e = qj * fd
                    cs = plsc.cumsum(e)
                    off_next = lane_shift(off)
                    ne = off != off_next
                    plus_m = ne | is15
                    plsc.addupdate_scatter(win_v, [off], cs, mask=plus_m)
                    plsc.addupdate_scatter(win_v, [off_next], -cs, mask=ne)

        def compute_rows_slow(b, nr, row0):
            pltpu.sync_copy(ii_hbm.at[pl.ds(row0, nr)], iis_v.at[0, pl.ds(0, nr)])

            def row_body(r, carry):
                for v in range(ROW // LANES):
                    sl = pl.ds(v * LANES, LANES)
                    pk = pk_v[b, r, sl]
                    fd = plsc.bitcast(pk & jnp.int32(-65536), jnp.float32)
                    qj = plsc.load_gather(qa_v, [jj_v[b, r, sl]])
                    e_v[0, r, sl] = qj * fd
                pltpu.sync_copy(e_v.at[0, r], acc_sh.at[iis_v.at[0, r]], add=True)
                return carry
            lax.fori_loop(0, nr, row_body, 0)

        def do_chunk(b, nr, row0):
            last = jnp.max(pk_v[b, nr - 1, pl.ds(ROW - LANES, LANES)] & 0xFFFF)
            fast = last < WIN

            @pl.when(fast)
            def _():
                compute_rows_fast(b, nr)

            @pl.when(jnp.logical_not(fast))
            def _():
                compute_rows_slow(b, nr, row0)

        def half_step(k, b):
            wait_stage(b)
            @pl.when(k < n_pipe - 1)
            def _():
                stage(row_base + (k + 1) * CHUNK_ROWS, 1 - b)
            do_chunk(b, CHUNK_ROWS, row_base + k * CHUNK_ROWS)

        if n_pipe > 0:
            def pipe_body(m, carry):
                half_step(2 * m, 0)
                half_step(2 * m + 1, 1)
                return carry
            lax.fori_loop(0, n_pipe // 2, pipe_body, 0)

        def process8(row0):
            pltpu.sync_copy(pk_hbm.at[pl.ds(row0, 8)], pk_v.at[0, pl.ds(0, 8)])
            pltpu.sync_copy(jj_hbm.at[pl.ds(row0, 8)], jj_v.at[0, pl.ds(0, 8)])
            do_chunk(0, 8, row0)

        if n_tail8 > 0:
            def tail_body(t, carry):
                process8(row_base + n_pipe * CHUNK_ROWS + t * 8)
                return carry
            lax.fori_loop(0, n_tail8, tail_body, 0)

        if n_extra8 > 0:
            @pl.when(wid < n_extra8)
            def _():
                process8(NW * base_rows + wid * 8)

        lo16 = plsc.load_gather(lo_v, [wid + jnp.zeros((LANES,), jnp.int32)])

        def flush_body(j, carry):
            base = lo16 + j * ROW
            for v in range(ROW // LANES):
                idx = base + (v * LANES) + iota16
                iis_v[0, 0, pl.ds(v * LANES, LANES)] = jnp.minimum(idx, n_nodes - 1)
            pltpu.sync_copy(win_v.at[pl.ds(j * ROW, ROW)],
                            acc_sh.at[iis_v.at[0, 0]], add=True)
            return carry
        lax.fori_loop(0, WIN // ROW, flush_body, 0)

        plsc.subcore_barrier()

        @pl.when(s < NS - 1)
        def _():
            pltpu.sync_copy(acc_sh.at[pl.ds(s * slc, slc)],
                            qa_v.at[pl.ds(0, slc)])
            pltpu.sync_copy(qa_v.at[pl.ds(0, slc)],
                            out_hbm.at[pl.ds(c * n_nodes + s * slc, slc)])

        @pl.when(s == NS - 1)
        def _():
            pltpu.sync_copy(acc_sh.at[pl.ds((NS - 1) * slc, last_slc)],
                            qa_v.at[pl.ds(0, last_slc)])
            pltpu.sync_copy(qa_v.at[pl.ds(0, last_slc)],
                            out_hbm.at[pl.ds(c * n_nodes + (NS - 1) * slc, last_slc)])

    return pl.kernel(
        body,
        out_type=jax.ShapeDtypeStruct((NC * n_nodes,), jnp.float32),
        mesh=mesh,
        compiler_params=pltpu.CompilerParams(needs_layout_passes=False),
        scratch_types=[
            pltpu.VMEM((n_nodes,), jnp.float32),
            pltpu.VMEM((2, CHUNK_ROWS, ROW), jnp.int32),
            pltpu.VMEM((2, CHUNK_ROWS, ROW), jnp.int32),
            pltpu.VMEM((1, CHUNK_ROWS, ROW), jnp.float32),
            pltpu.VMEM((1, CHUNK_ROWS, ROW), jnp.int32),
            pltpu.VMEM((NW,), jnp.int32),
            pltpu.VMEM((WIN,), jnp.float32),
            pltpu.VMEM_SHARED((n_nodes,), jnp.float32),
            pltpu.SemaphoreType.DMA,
        ],
    )


_sc_kernel = _build_sc_kernel(N_NODES, N_ROWS)

_row_tile = np.zeros((N_ROWS,), np.int32)
for _w in range(NW):
    _row_tile[_w * BASE_ROWS:(_w + 1) * BASE_ROWS] = _w
for _w in range(N_EXTRA8):
    _row_tile[NW * BASE_ROWS + _w * 8: NW * BASE_ROWS + (_w + 1) * 8] = _w
_tile_base_rows = np.arange(NW, dtype=np.int32) * BASE_ROWS

_PACK_ROWS = 1000


def _combine_body(p_ref, qa_ref, o_ref):
    o_ref[...] = (KEHALF * qa_ref[...]) * (p_ref[0, :] + p_ref[1, :])


def kernel(Dij, Qa, idx_i, idx_j):
    d2 = Dij.reshape(N_ROWS, ROW)
    ii2 = idx_i.reshape(N_ROWS, ROW)
    jj2 = idx_j.reshape(N_ROWS, ROW)
    lo_tile = ii2[_tile_base_rows, 0]
    lo_row = lo_tile[_row_tile].reshape(N_ROWS, 1)
    pk = pl.pallas_call(
        _pack_body,
        grid=(N_ROWS // _PACK_ROWS,),
        in_specs=[pl.BlockSpec((_PACK_ROWS, ROW), lambda i: (i, 0)),
                  pl.BlockSpec((_PACK_ROWS, ROW), lambda i: (i, 0)),
                  pl.BlockSpec((_PACK_ROWS, 1), lambda i: (i, 0))],
        out_specs=pl.BlockSpec((_PACK_ROWS, ROW), lambda i: (i, 0)),
        out_shape=jax.ShapeDtypeStruct((N_ROWS, ROW), jnp.int32),
    )(d2, ii2, lo_row)
    partial = _sc_kernel(pk, Qa, ii2, jj2, lo_tile).reshape(NC, N_NODES)
    out = pl.pallas_call(
        _combine_body,
        out_shape=jax.ShapeDtypeStruct((N_NODES,), jnp.float32),
    )(partial, Qa)
    return out


# speedup vs baseline: 1.2598x; 1.2598x over previous
import functools

import jax
import jax.numpy as jnp
from jax import lax
from jax.experimental import pallas as pl
from jax.experimental.pallas import tpu as pltpu
from jax.experimental.pallas import tpu_sc as plsc

KEHALF = 7.199822675975274
SR_CUTOFF = 5.0
LR_CUTOFF = 10.0
LR_CUTOFF2 = LR_CUTOFF * LR_CUTOFF

N_NODES = 100000
N_EDGES = 6400000

LANES = 16
ROW = 128
NC = 2
NS = 16
NW = NC * NS
CHUNK_ROWS = 16
WIN = 10240


def _dist_energy(d):
    x = d * d + 1.0
    y = lax.rsqrt(x)
    c2 = 1.0 / LR_CUTOFF2
    c1 = 2.0 / LR_CUTOFF
    e_ord = (1.0 / d - c1) + d * c2
    e_shl = y * (1.0 + x * c2) - c1
    t = jnp.minimum(d * (2.0 / SR_CUTOFF), 1.0)
    t3 = (t * t) * t
    q = (6.0 * t - 15.0) * t + 10.0
    diff = e_shl - e_ord
    e = e_shl - (t3 * q) * diff
    return jnp.where(d <= LR_CUTOFF, e, 0.0)


def _dist_body(d_ref, o_ref):
    o_ref[...] = _dist_energy(d_ref[...])


def _build_sc_kernel(n_nodes, n_rows):
    assert n_rows % 8 == 0
    base_rows = (n_rows // NW) // 8 * 8
    leftover = n_rows - NW * base_rows
    assert leftover % 8 == 0
    n_extra8 = leftover // 8
    assert n_extra8 <= NW
    n_full = base_rows // CHUNK_ROWS
    n_pipe = n_full - (n_full % 2)
    tail_rows = base_rows - n_pipe * CHUNK_ROWS
    assert tail_rows % 8 == 0
    n_tail8 = tail_rows // 8
    slc = ((n_nodes + NS - 1) // NS + 7) // 8 * 8
    last_slc = n_nodes - (NS - 1) * slc
    assert last_slc > 0

    mesh = plsc.VectorSubcoreMesh(core_axis_name="c", subcore_axis_name="s")

    def body(d_hbm, qa_hbm, ii_hbm, jj_hbm, out_hbm,
             qa_v, d_v, ii_v, jj_v, e_v, win_v, acc_sh, sem_in, sem_scat):
        c = lax.axis_index("c")
        s = lax.axis_index("s")
        wid = s * NC + c
        row_base = wid * base_rows
        iota16 = lax.broadcasted_iota(jnp.int32, (LANES,), 0)
        is15 = iota16 == (LANES - 1)
        shift1 = jnp.minimum(iota16 + 1, LANES - 1)
        zeros16 = jnp.zeros((LANES,), jnp.float32)

        def stage(row0, b):
            pltpu.async_copy(d_hbm.at[pl.ds(row0, CHUNK_ROWS)], d_v.at[b], sem_in)
            pltpu.async_copy(ii_hbm.at[pl.ds(row0, CHUNK_ROWS)], ii_v.at[b], sem_in)
            pltpu.async_copy(jj_hbm.at[pl.ds(row0, CHUNK_ROWS)], jj_v.at[b], sem_in)

        def wait_stage(b):
            pltpu.make_async_copy(d_hbm.at[pl.ds(0, CHUNK_ROWS)], d_v.at[b], sem_in).wait()
            pltpu.make_async_copy(ii_hbm.at[pl.ds(0, CHUNK_ROWS)], ii_v.at[b], sem_in).wait()
            pltpu.make_async_copy(jj_hbm.at[pl.ds(0, CHUNK_ROWS)], jj_v.at[b], sem_in).wait()

        if n_pipe > 0:
            stage(row_base, 0)

        def zwin_body(i, carry):
            win_v[pl.ds(i * LANES, LANES)] = zeros16
            return carry
        lax.fori_loop(0, WIN // LANES, zwin_body, 0)

        def zero_body(i, carry):
            qa_v[pl.ds(i * LANES, LANES)] = zeros16
            return carry
        lax.fori_loop(0, slc // LANES + 1, zero_body, 0)

        @pl.when(s < NS - 1)
        def _():
            pltpu.sync_copy(qa_v.at[pl.ds(0, slc)],
                            acc_sh.at[pl.ds(s * slc, slc)])

        @pl.when(s == NS - 1)
        def _():
            pltpu.sync_copy(qa_v.at[pl.ds(0, last_slc)],
                            acc_sh.at[pl.ds((NS - 1) * slc, last_slc)])

        pltpu.sync_copy(qa_hbm, qa_v)
        plsc.subcore_barrier()

        def compute_rows_fast(b, nr, lo):
            @plsc.parallel_loop(0, nr, unroll=2)
            def _(r):
                for v in range(ROW // LANES):
                    sl = pl.ds(v * LANES, LANES)
                    ep = d_v[b, r, sl]
                    ii = ii_v[b, r, sl]
                    qj = plsc.load_gather(qa_v, [jj_v[b, r, sl]])
                    e = qj * ep
                    cs = plsc.cumsum(e)
                    ii_next = lax.gather(
                        ii, shift1[:, None],
                        lax.GatherDimensionNumbers(
                            offset_dims=(), collapsed_slice_dims=(0,),
                            start_index_map=(0,)),
                        slice_sizes=(1,),
                        mode=lax.GatherScatterMode.PROMISE_IN_BOUNDS)
                    ne = ii != ii_next
                    plus_m = ne | is15
                    off = ii - lo
                    off_next = ii_next - lo
                    plsc.addupdate_scatter(win_v, [off], cs, mask=plus_m)
                    plsc.addupdate_scatter(win_v, [off_next], -cs, mask=ne)

        def compute_rows_slow(b, nr):
            def row_body(r, carry):
                for v in range(ROW // LANES):
                    sl = pl.ds(v * LANES, LANES)
                    ep = d_v[b, r, sl]
                    qj = plsc.load_gather(qa_v, [jj_v[b, r, sl]])
                    e_v[0, r, sl] = qj * ep
                pltpu.sync_copy(e_v.at[0, r], acc_sh.at[ii_v.at[b, r]], add=True)
                return carry
            lax.fori_loop(0, nr, row_body, 0)

        def do_chunk(b, nr, lo):
            last = jnp.max(ii_v[b, nr - 1, pl.ds(ROW - LANES, LANES)])
            fast = (last - lo) < WIN

            @pl.when(fast)
            def _():
                compute_rows_fast(b, nr, lo)

            @pl.when(jnp.logical_not(fast))
            def _():
                compute_rows_slow(b, nr)

        def half_step(k, b, m, lo):
            wait_stage(b)
            if b == 0:
                cand = jnp.min(ii_v[0, 0, pl.ds(0, LANES)])
                lo = jnp.where(m == 0, cand, lo)
            @pl.when(k < n_pipe - 1)
            def _():
                stage(row_base + (k + 1) * CHUNK_ROWS, 1 - b)
            do_chunk(b, CHUNK_ROWS, lo)
            return lo

        lo = jnp.int32(0)
        if n_pipe > 0:
            def pipe_body(m, lo):
                lo = half_step(2 * m, 0, m, lo)
                lo = half_step(2 * m + 1, 1, m, lo)
                return lo
            lo = lax.fori_loop(0, n_pipe // 2, pipe_body, lo)

        def process8(row0, lo):
            pltpu.sync_copy(d_hbm.at[pl.ds(row0, 8)], d_v.at[0, pl.ds(0, 8)])
            pltpu.sync_copy(ii_hbm.at[pl.ds(row0, 8)], ii_v.at[0, pl.ds(0, 8)])
            pltpu.sync_copy(jj_hbm.at[pl.ds(row0, 8)], jj_v.at[0, pl.ds(0, 8)])
            do_chunk(0, 8, lo)

        if n_tail8 > 0:
            def tail_body(t, carry):
                process8(row_base + n_pipe * CHUNK_ROWS + t * 8, lo)
                return carry
            lax.fori_loop(0, n_tail8, tail_body, 0)

        if n_extra8 > 0:
            @pl.when(wid < n_extra8)
            def _():
                process8(NW * base_rows + wid * 8, lo)

        def flush_body(j, carry):
            base = lo + j * ROW
            for v in range(ROW // LANES):
                idx = base + (v * LANES) + iota16
                ii_v[0, 0, pl.ds(v * LANES, LANES)] = jnp.minimum(idx, n_nodes - 1)
            pltpu.sync_copy(win_v.at[pl.ds(j * ROW, ROW)],
                            acc_sh.at[ii_v.at[0, 0]], add=True)
            return carry
        lax.fori_loop(0, WIN // ROW, flush_body, 0)

        plsc.subcore_barrier()

        @pl.when(s < NS - 1)
        def _():
            pltpu.sync_copy(acc_sh.at[pl.ds(s * slc, slc)],
                            qa_v.at[pl.ds(0, slc)])
            pltpu.sync_copy(qa_v.at[pl.ds(0, slc)],
                            out_hbm.at[pl.ds(c * n_nodes + s * slc, slc)])

        @pl.when(s == NS - 1)
        def _():
            pltpu.sync_copy(acc_sh.at[pl.ds((NS - 1) * slc, last_slc)],
                            qa_v.at[pl.ds(0, last_slc)])
            pltpu.sync_copy(qa_v.at[pl.ds(0, last_slc)],
                            out_hbm.at[pl.ds(c * n_nodes + (NS - 1) * slc, last_slc)])

    return pl.kernel(
        body,
        out_type=jax.ShapeDtypeStruct((NC * n_nodes,), jnp.float32),
        mesh=mesh,
        compiler_params=pltpu.CompilerParams(needs_layout_passes=False),
        scratch_types=[
            pltpu.VMEM((n_nodes,), jnp.float32),
            pltpu.VMEM((2, CHUNK_ROWS, ROW), jnp.float32),
            pltpu.VMEM((2, CHUNK_ROWS, ROW), jnp.int32),
            pltpu.VMEM((2, CHUNK_ROWS, ROW), jnp.int32),
            pltpu.VMEM((1, CHUNK_ROWS, ROW), jnp.float32),
            pltpu.VMEM((WIN,), jnp.float32),
            pltpu.VMEM_SHARED((n_nodes,), jnp.float32),
            pltpu.SemaphoreType.DMA,
            pltpu.SemaphoreType.DMA,
        ],
    )


_sc_kernel = _build_sc_kernel(N_NODES, N_EDGES // ROW)


def _combine_body(p_ref, qa_ref, o_ref):
    o_ref[...] = (KEHALF * qa_ref[...]) * (p_ref[0, :] + p_ref[1, :])


_DIST_ROWS = 1000


def kernel(Dij, Qa, idx_i, idx_j):
    n_rows = N_EDGES // ROW
    d2 = Dij.reshape(n_rows, ROW)
    ii2 = idx_i.reshape(n_rows, ROW)
    jj2 = idx_j.reshape(n_rows, ROW)
    fd = pl.pallas_call(
        _dist_body,
        grid=(n_rows // _DIST_ROWS,),
        in_specs=[pl.BlockSpec((_DIST_ROWS, ROW), lambda i: (i, 0))],
        out_specs=pl.BlockSpec((_DIST_ROWS, ROW), lambda i: (i, 0)),
        out_shape=jax.ShapeDtypeStruct((n_rows, ROW), jnp.float32),
    )(d2)
    partial = _sc_kernel(fd, Qa, ii2, jj2).reshape(NC, N_NODES)
    out = pl.pallas_call(
        _combine_body,
        out_shape=jax.ShapeDtypeStruct((N_NODES,), jnp.float32),
    )(partial, Qa)
    return out
